# baseline (speedup 1.0000x reference)
import functools

import jax
import jax.numpy as jnp
from jax import lax
from jax.experimental import pallas as pl
from jax.experimental.pallas import tpu as pltpu
from jax.experimental.pallas import tpu_sc as plsc

E = 8
K = 2
D_IN = 1024
D_H = 2048
D_OUT = 1024
N = 4096

BLK = 256
NB = 40
CAP = NB * BLK
NW = 32

_GATE_BLK = 512


def _gate_body(x_ref, wg_ref, bg_ref, idx_ref, wts_ref):
    s = lax.dot_general(
        x_ref[...].astype(jnp.bfloat16),
        wg_ref[...].astype(jnp.bfloat16),
        (((1,), (0,)), ((), ())),
        preferred_element_type=jnp.float32,
    ) + bg_ref[...]
    lanes = lax.broadcasted_iota(jnp.int32, (_GATE_BLK, 128), 1)
    neg = jnp.float32(-1e30)
    s = jnp.where(lanes < E, s, neg)
    m1 = jnp.max(s, axis=1, keepdims=True)
    i1 = jnp.min(jnp.where(s == m1, lanes, 127), axis=1, keepdims=True)
    s2 = jnp.where(lanes == i1, neg, s)
    m2 = jnp.max(s2, axis=1, keepdims=True)
    i2 = jnp.min(jnp.where(s2 == m2, lanes, 127), axis=1, keepdims=True)
    ex = jnp.exp(s - m1)
    z = jnp.sum(ex, axis=1, keepdims=True)
    p1 = 1.0 / z
    p2 = jnp.exp(m2 - m1) / z
    denom = p1 + p2 + jnp.float32(1e-8)
    w1 = p1 / denom
    w2 = p2 / denom
    idx_ref[...] = jnp.where(lanes == 0, i1, jnp.where(lanes == 1, i2, 0))
    wts_ref[...] = jnp.where(lanes == 0, w1, jnp.where(lanes == 1, w2, 0.0))


def _gate(x, wg_t_pad, bg_pad):
    return pl.pallas_call(
        _gate_body,
        grid=(N // _GATE_BLK,),
        in_specs=[
            pl.BlockSpec((_GATE_BLK, D_IN), lambda i: (i, 0)),
            pl.BlockSpec((D_IN, 128), lambda i: (0, 0)),
            pl.BlockSpec((1, 128), lambda i: (0, 0)),
        ],
        out_specs=[
            pl.BlockSpec((_GATE_BLK, 128), lambda i: (i, 0)),
            pl.BlockSpec((_GATE_BLK, 128), lambda i: (i, 0)),
        ],
        out_shape=[
            jax.ShapeDtypeStruct((N, 128), jnp.int32),
            jax.ShapeDtypeStruct((N, 128), jnp.float32),
        ],
    )(x, wg_t_pad, bg_pad)


def _mlp_body(be_ref, bv_ref, xg_ref, w1_ref, b1_ref, w2_ref, b2_ref,
              w3_ref, b3_ref, ws_ref, out_ref):
    i = pl.program_id(0)

    @pl.when(bv_ref[i] == 1)
    def _():
        xb = xg_ref[...].astype(jnp.bfloat16)
        h1 = lax.dot_general(
            xb, w1_ref[0], (((1,), (0,)), ((), ())),
            preferred_element_type=jnp.float32)
        h1 = jnp.maximum(h1 + b1_ref[0], 0.0).astype(jnp.bfloat16)
        h2 = lax.dot_general(
            h1, w2_ref[0], (((1,), (0,)), ((), ())),
            preferred_element_type=jnp.float32)
        h2 = jnp.maximum(h2 + b2_ref[0], 0.0).astype(jnp.bfloat16)
        o = lax.dot_general(
            h2, w3_ref[0], (((1,), (0,)), ((), ())),
            preferred_element_type=jnp.float32)
        out_ref[...] = (o + b3_ref[0]) * ws_ref[...]


def _mlp(xg, w1t, b1r, w2t, b2r, w3t, b3r, wslot, block_expert, block_valid):
    grid_spec = pltpu.PrefetchScalarGridSpec(
        num_scalar_prefetch=2,
        grid=(NB,),
        in_specs=[
            pl.BlockSpec((BLK, D_IN), lambda i, be, bv: (i, 0)),
            pl.BlockSpec((1, D_IN, D_H), lambda i, be, bv: (be[i], 0, 0)),
            pl.BlockSpec((1, 1, D_H), lambda i, be, bv: (be[i], 0, 0)),
            pl.BlockSpec((1, D_H, D_H), lambda i, be, bv: (be[i], 0, 0)),
            pl.BlockSpec((1, 1, D_H), lambda i, be, bv: (be[i], 0, 0)),
            pl.BlockSpec((1, D_H, D_OUT), lambda i, be, bv: (be[i], 0, 0)),
            pl.BlockSpec((1, 1, D_OUT), lambda i, be, bv: (be[i], 0, 0)),
            pl.BlockSpec((BLK, 1), lambda i, be, bv: (i, 0)),
        ],
        out_specs=pl.BlockSpec((BLK, D_OUT), lambda i, be, bv: (i, 0)),
    )
    return pl.pallas_call(
        _mlp_body,
        grid_spec=grid_spec,
        out_shape=jax.ShapeDtypeStruct((CAP, D_OUT), jnp.float32),
    )(block_expert, block_valid, xg, w1t, b1r, w2t, b2r, w3t, b3r, wslot)


def _sc_gather(x, row_ids):
    rpw = CAP // NW
    ch = 64
    nch = rpw // ch
    mesh = plsc.VectorSubcoreMesh(core_axis_name="c", subcore_axis_name="s")

    @functools.partial(
        pl.kernel,
        mesh=mesh,
        out_type=jax.ShapeDtypeStruct((CAP, D_IN), jnp.float32),
        scratch_types=[
            pltpu.VMEM((ch,), jnp.int32),
            pltpu.VMEM((ch, D_IN), jnp.float32),
            pltpu.SemaphoreType.DMA,
        ],
    )
    def k(x_hbm, ids_hbm, out_hbm, idx_v, rows_v, sem):
        wid = lax.axis_index("s") * 2 + lax.axis_index("c")
        base = wid * rpw

        def body(c, _):
            off = base + c * ch
            pltpu.sync_copy(ids_hbm.at[pl.ds(off, ch)], idx_v)
            pltpu.async_copy(x_hbm.at[idx_v], rows_v, sem).wait()
            pltpu.sync_copy(rows_v, out_hbm.at[pl.ds(off, ch)])
            return 0

        lax.fori_loop(0, nch, body, 0)

    return k(x, row_ids)


def _sc_combine(outw, pos1, pos2):
    tpw = N // NW
    ch = 32
    nch = tpw // ch
    mesh = plsc.VectorSubcoreMesh(core_axis_name="c", subcore_axis_name="s")

    @functools.partial(
        pl.kernel,
        mesh=mesh,
        out_type=jax.ShapeDtypeStruct((N, D_OUT), jnp.float32),
        scratch_types=[
            pltpu.VMEM((ch,), jnp.int32),
            pltpu.VMEM((ch,), jnp.int32),
            pltpu.VMEM((ch, D_OUT), jnp.float32),
            pltpu.VMEM((ch, D_OUT), jnp.float32),
            pltpu.VMEM((ch, D_OUT), jnp.float32),
            pltpu.SemaphoreType.DMA,
            pltpu.SemaphoreType.DMA,
        ],
    )
    def k(ow_hbm, p1_hbm, p2_hbm, y_hbm, i1_v, i2_v, r1_v, r2_v, y_v,
          sem1, sem2):
        wid = lax.axis_index("s") * 2 + lax.axis_index("c")
        base = wid * tpw

        def body(c, _):
            off = base + c * ch
            pltpu.sync_copy(p1_hbm.at[pl.ds(off, ch)], i1_v)
            pltpu.sync_copy(p2_hbm.at[pl.ds(off, ch)], i2_v)
            cp1 = pltpu.async_copy(ow_hbm.at[i1_v], r1_v, sem1)
            cp2 = pltpu.async_copy(ow_hbm.at[i2_v], r2_v, sem2)
            cp1.wait()
            cp2.wait()

            def tok(t, _):
                def lane(j, _):
                    sl = pl.ds(j * 16, 16)
                    y_v[t, sl] = r1_v[t, sl] + r2_v[t, sl]
                    return 0

                lax.fori_loop(0, D_OUT // 16, lane, 0)
                return 0

            lax.fori_loop(0, ch, tok, 0)
            pltpu.sync_copy(y_v, y_hbm.at[pl.ds(off, ch)])
            return 0

        lax.fori_loop(0, nch, body, 0)

    return k(outw, pos1, pos2)


def kernel(x, W1, b1, W2, b2, W3, b3, Wg, bg):
    f32 = jnp.float32

    w1t = jnp.transpose(W1, (0, 2, 1)).astype(jnp.bfloat16)
    w2t = jnp.transpose(W2, (0, 2, 1)).astype(jnp.bfloat16)
    w3t = jnp.transpose(W3, (0, 2, 1)).astype(jnp.bfloat16)
    b1r = b1.reshape(E, 1, D_H)
    b2r = b2.reshape(E, 1, D_H)
    b3r = b3.reshape(E, 1, D_OUT)
    wg_t_pad = jnp.zeros((D_IN, 128), f32).at[:, :E].set(Wg.T)
    bg_pad = jnp.zeros((1, 128), f32).at[0, :E].set(bg)

    idx2, wts2 = _gate(x, wg_t_pad, bg_pad)
    i1 = idx2[:, 0]
    i2 = idx2[:, 1]
    wv1 = wts2[:, 0]
    wv2 = wts2[:, 1]

    oh = (jax.nn.one_hot(i1, E, dtype=jnp.int32)
          + jax.nn.one_hot(i2, E, dtype=jnp.int32))
    cum = jnp.cumsum(oh, axis=0)
    counts = cum[-1]
    bpe = (counts + BLK - 1) // BLK
    bstart = jnp.concatenate([jnp.zeros((1,), jnp.int32),
                              jnp.cumsum(bpe).astype(jnp.int32)])
    offset = bstart[:E] * BLK
    nbu = bstart[E]

    c1 = jnp.take_along_axis(cum, i1[:, None], axis=1)[:, 0]
    c2 = jnp.take_along_axis(cum, i2[:, None], axis=1)[:, 0]
    pos1 = (offset[i1] + c1 - 1).astype(jnp.int32)
    pos2 = (offset[i2] + c2 - 1).astype(jnp.int32)

    tok = jnp.arange(N, dtype=jnp.int32)
    row_ids = (jnp.zeros((CAP,), jnp.int32)
               .at[pos1].set(tok, unique_indices=True)
               .at[pos2].set(tok, unique_indices=True))
    wslot = (jnp.zeros((CAP,), f32)
             .at[pos1].set(wv1, unique_indices=True)
             .at[pos2].set(wv2, unique_indices=True)).reshape(CAP, 1)

    bids = jnp.arange(NB, dtype=jnp.int32)
    be = jnp.searchsorted(bstart[1:], bids, side="right").astype(jnp.int32)
    be = jnp.minimum(be, E - 1)
    last_e = be[jnp.maximum(nbu - 1, 0)]
    be = jnp.where(bids < nbu, be, last_e)
    bvalid = (bids < nbu).astype(jnp.int32)

    xg = _sc_gather(x, row_ids)

    outw = _mlp(xg, w1t, b1r, w2t, b2r, w3t, b3r, wslot, be, bvalid)

    return _sc_combine(outw, pos1, pos2)
